# Initial kernel scaffold; baseline (speedup 1.0000x reference)
import jax
import jax.numpy as jnp
from jax.experimental import pallas as pl


def kernel(X1, X2, A, W):
    raise NotImplementedError("write your pallas kernel here")


# speedup vs baseline: 49.5452x; 49.5452x over previous
import functools

import jax
import jax.numpy as jnp
import numpy as np
from jax.experimental import pallas as pl
from jax.experimental.pallas import tpu as pltpu

_DEPTH = 10
_NB_NODES = 2 ** (_DEPTH + 1) - 1
_NB_SPLIT = 2 ** _DEPTH - 1

_LVL = [np.array([0], dtype=np.int64)]
for _d in range(_DEPTH):
    _LVL.append(np.concatenate([2 * _LVL[_d] + 1, 2 * _LVL[_d] + 2]))

_XA_NODES = np.full(1024, -1, dtype=np.int64)
for _d in range(7):
    _XA_NODES[2 ** _d - 1: 2 ** (_d + 1) - 1] = _LVL[_d]
_XA_NODES[128:256] = _LVL[7]
_XA_NODES[256:512] = _LVL[8]
_XA_NODES[512:1024] = _LVL[9]

_Z_NODES = np.full(2048, -1, dtype=np.int64)
for _d in range(7):
    _Z_NODES[2 ** _d - 1: 2 ** (_d + 1) - 1] = _LVL[_d]
_Z_NODES[128:256] = _LVL[7]
_Z_NODES[256:512] = _LVL[8]
_Z_NODES[512:1024] = _LVL[9]
_Z_NODES[1024:2048] = _LVL[10]

_XA_VALID = _XA_NODES >= 0
_XA_IDX = np.where(_XA_VALID, _XA_NODES, 0)
_Z_VALID = _Z_NODES >= 0
_Z_IDX = np.where(_Z_VALID, _Z_NODES, 0)

_BT = 256


def _rot(x, k):
    return pltpu.roll(x, (-k) % x.shape[1], 1)


def _clip01(x):
    return jnp.clip(x, 0.0, 1.0)


def _tree_kernel(x1_ref, x2_ref, ap_ref, ab_ref, w1t_ref, wzt_ref, wb_ref,
                 out_ref):
    f32 = jnp.float32
    hi = jax.lax.Precision.HIGHEST
    bt = x1_ref.shape[0]

    xa = jax.lax.dot_general(
        x2_ref[...], ap_ref[...], (((1,), (1,)), ((), ())),
        preferred_element_type=f32, precision=hi) + ab_ref[...]

    xas = xa[:, 0:128]
    lane = jax.lax.broadcasted_iota(jnp.int32, (bt, 128), 1)

    u = jnp.ones((bt, 128), f32)
    s_group = jnp.zeros((bt, 128), f32)
    for d in range(7):
        n = 2 ** d
        s_group = jnp.where((lane >= n - 1) & (lane < 2 * n - 1),
                            _rot(u, 1), s_group)
        xt = _rot(xas, n - 1)
        for s in range(d, 7):
            m = 2 ** s
            xt = jnp.where(lane % (2 * m) >= m, _rot(xt, -m), xt)
        signed = jnp.where((lane >> d) % 2 == 1, -xt, xt)
        u = jnp.minimum(u, _clip01(signed))
    q7 = u

    xa7 = xa[:, 128:256]
    xa8 = xa[:, 256:512]
    xa9 = xa[:, 512:1024]
    q8 = jnp.concatenate(
        [jnp.minimum(q7, _clip01(xa7)), jnp.minimum(q7, _clip01(-xa7))], axis=1)
    q9 = jnp.concatenate(
        [jnp.minimum(q8, _clip01(xa8)), jnp.minimum(q8, _clip01(-xa8))], axis=1)
    q10 = jnp.concatenate(
        [jnp.minimum(q9, _clip01(xa9)), jnp.minimum(q9, _clip01(-xa9))], axis=1)
    z = jnp.concatenate([s_group, q7, q8, q9, q10], axis=1)

    out = jax.lax.dot_general(
        x1_ref[...], w1t_ref[...], (((1,), (0,)), ((), ())),
        preferred_element_type=f32, precision=hi) + wb_ref[...]
    out += jax.lax.dot_general(
        z, wzt_ref[...], (((1,), (0,)), ((), ())),
        preferred_element_type=f32, precision=hi)
    out_ref[...] = out


@functools.partial(jax.jit, static_argnames=("interpret",))
def kernel(X1, X2, A, W, interpret=False):
    batch, in1 = X1.shape
    out_dim = W.shape[0]
    f32 = jnp.float32

    a_main = A[:, :-1]
    a_bias = A[:, -1]
    xa_valid = jnp.asarray(_XA_VALID)
    ap = jnp.where(xa_valid[:, None], a_main[_XA_IDX], 0.0).astype(f32)
    ab = jnp.where(xa_valid, a_bias[_XA_IDX], 0.0).astype(f32)[None, :]

    w1t = W[:, :in1].T.astype(f32)
    wb = W[:, in1][None, :].astype(f32)
    wz = W[:, in1 + 1:]
    z_valid = jnp.asarray(_Z_VALID)
    wzt = jnp.where(z_valid[:, None], wz[:, _Z_IDX].T, 0.0).astype(f32)

    grid = (batch // _BT,)
    out = pl.pallas_call(
        _tree_kernel,
        grid=grid,
        in_specs=[
            pl.BlockSpec((_BT, in1), lambda i: (i, 0)),
            pl.BlockSpec((_BT, X2.shape[1]), lambda i: (i, 0)),
            pl.BlockSpec((1024, X2.shape[1]), lambda i: (0, 0)),
            pl.BlockSpec((1, 1024), lambda i: (0, 0)),
            pl.BlockSpec((in1, out_dim), lambda i: (0, 0)),
            pl.BlockSpec((2048, out_dim), lambda i: (0, 0)),
            pl.BlockSpec((1, out_dim), lambda i: (0, 0)),
        ],
        out_specs=pl.BlockSpec((_BT, out_dim), lambda i: (i, 0)),
        out_shape=jax.ShapeDtypeStruct((batch, out_dim), f32),
        compiler_params=pltpu.CompilerParams(
            dimension_semantics=("parallel",)),
        interpret=interpret,
    )(X1.astype(f32), X2.astype(f32), ap, ab, w1t, wzt, wb)
    return out


# speedup vs baseline: 104.9548x; 2.1184x over previous
import functools

import jax
import jax.numpy as jnp
import numpy as np
from jax.experimental import pallas as pl
from jax.experimental.pallas import tpu as pltpu

_DEPTH = 10

_LVL = [np.array([0], dtype=np.int64)]
for _d in range(_DEPTH):
    _LVL.append(np.concatenate([2 * _LVL[_d] + 1, 2 * _LVL[_d] + 2]))

_XA_NODES = np.full(1024, -1, dtype=np.int64)
for _d in range(7):
    _XA_NODES[2 ** _d - 1: 2 ** (_d + 1) - 1] = _LVL[_d]
_XA_NODES[128:256] = _LVL[7]
_XA_NODES[256:512] = _LVL[8]
_XA_NODES[512:1024] = _LVL[9]
_XA_ROW_IDX = np.where(_XA_NODES >= 0, _XA_NODES, 2 ** _DEPTH - 1).astype(np.int32)
_XA_PAD = int(np.argwhere(_XA_NODES < 0)[0, 0])

_Z_NODES = np.full(2048, -1, dtype=np.int64)
for _d in range(7):
    _Z_NODES[2 ** _d - 1: 2 ** (_d + 1) - 1] = _LVL[_d]
_Z_NODES[128:256] = _LVL[7]
_Z_NODES[256:512] = _LVL[8]
_Z_NODES[512:1024] = _LVL[9]
_Z_NODES[1024:2048] = _LVL[10]
_W_COL_IDX = np.where(_Z_NODES >= 0, 513 + _Z_NODES, 512).astype(np.int32)

_BT = 256


def _rot(x, k):
    return pltpu.roll(x, (-k) % x.shape[1], 1)


def _clip01(x):
    return jnp.clip(x, 0.0, 1.0)


def _tree_kernel(x1_ref, x2_ref, ap_ref, ab_ref, w_ref, wz_ref, out_ref):
    f32 = jnp.float32
    bt = x1_ref.shape[0]

    xa = jax.lax.dot_general(
        x2_ref[...], ap_ref[:, 0:512], (((1,), (1,)), ((), ())),
        preferred_element_type=f32) + ab_ref[...]

    xas = xa[:, 0:128]
    lane = jax.lax.broadcasted_iota(jnp.int32, (bt, 128), 1)
    bit = [(lane & (1 << s)) != 0 for s in range(7)]
    lvl = [(lane >= 2 ** d - 1) & (lane < 2 ** (d + 1) - 1) for d in range(7)]

    u = jnp.ones((bt, 128), f32)
    s_group = jnp.zeros((bt, 128), f32)
    for d in range(7):
        n = 2 ** d
        s_group = jnp.where(lvl[d], _rot(u, 1), s_group)
        xt = _rot(xas, n - 1)
        for s in range(d, 7):
            xt = jnp.where(bit[s], _rot(xt, -(2 ** s)), xt)
        signed = jnp.where(bit[d], -xt, xt)
        u = jnp.minimum(u, _clip01(signed))
    s_group = jnp.where(lane == 127, 1.0, s_group)
    q7 = u

    xa7 = xa[:, 128:256]
    xa8 = xa[:, 256:512]
    xa9 = xa[:, 512:1024]
    q8 = jnp.concatenate(
        [jnp.minimum(q7, _clip01(xa7)), jnp.minimum(q7, _clip01(-xa7))], axis=1)
    q9 = jnp.concatenate(
        [jnp.minimum(q8, _clip01(xa8)), jnp.minimum(q8, _clip01(-xa8))], axis=1)
    q10 = jnp.concatenate(
        [jnp.minimum(q9, _clip01(xa9)), jnp.minimum(q9, _clip01(-xa9))], axis=1)
    z = jnp.concatenate([s_group, q7, q8, q9, q10], axis=1)

    out = jax.lax.dot_general(
        x1_ref[...], w_ref[:, 0:512], (((1,), (1,)), ((), ())),
        preferred_element_type=f32)
    out += jax.lax.dot_general(
        z, wz_ref[...], (((1,), (1,)), ((), ())),
        preferred_element_type=f32)
    out_ref[...] = out


@jax.jit
def kernel(X1, X2, A, W):
    batch, in1 = X1.shape
    out_dim = W.shape[0]
    f32 = jnp.float32

    ap = jnp.take(A.astype(f32), _XA_ROW_IDX, axis=0, mode="fill",
                  fill_value=0.0)
    ab = jnp.take(A[:, in1].astype(f32), _XA_ROW_IDX, mode="fill",
                  fill_value=0.0)[None, :]
    wzb = jnp.take(W.astype(f32), _W_COL_IDX, axis=1)

    grid = (batch // _BT,)
    out = pl.pallas_call(
        _tree_kernel,
        grid=grid,
        in_specs=[
            pl.BlockSpec((_BT, in1), lambda i: (i, 0)),
            pl.BlockSpec((_BT, X2.shape[1]), lambda i: (i, 0)),
            pl.BlockSpec((1024, A.shape[1]), lambda i: (0, 0)),
            pl.BlockSpec((1, 1024), lambda i: (0, 0)),
            pl.BlockSpec((out_dim, W.shape[1]), lambda i: (0, 0)),
            pl.BlockSpec((out_dim, 2048), lambda i: (0, 0)),
        ],
        out_specs=pl.BlockSpec((_BT, out_dim), lambda i: (i, 0)),
        out_shape=jax.ShapeDtypeStruct((batch, out_dim), f32),
        compiler_params=pltpu.CompilerParams(
            dimension_semantics=("parallel",)),
    )(X1.astype(f32), X2.astype(f32), ap, ab, W.astype(f32), wzb)
    return out


# speedup vs baseline: 136.7059x; 1.3025x over previous
import functools

import jax
import jax.numpy as jnp
import numpy as np
from jax.experimental import pallas as pl
from jax.experimental.pallas import tpu as pltpu

_DEPTH = 10

_LVL = [np.array([0], dtype=np.int64)]
for _d in range(_DEPTH):
    _LVL.append(np.concatenate([2 * _LVL[_d] + 1, 2 * _LVL[_d] + 2]))

_XA_NODES = np.empty(1792, dtype=np.int64)
for _d in range(7):
    _XA_NODES[_d * 128:(_d + 1) * 128] = _LVL[_d][np.arange(128) % (2 ** _d)]
_XA_NODES[896:1024] = _LVL[7]
_XA_NODES[1024:1280] = _LVL[8]
_XA_NODES[1280:1792] = _LVL[9]
_XA_ROW_IDX = _XA_NODES.astype(np.int32)

_Z_NODES = np.full(2048, -1, dtype=np.int64)
for _d in range(7):
    _Z_NODES[2 ** _d - 1: 2 ** (_d + 1) - 1] = _LVL[_d]
_Z_NODES[128:256] = _LVL[7]
_Z_NODES[256:512] = _LVL[8]
_Z_NODES[512:1024] = _LVL[9]
_Z_NODES[1024:2048] = _LVL[10]
_W_COL_IDX = np.where(_Z_NODES >= 0, 513 + _Z_NODES, 512).astype(np.int32)

_BT = 256


def _rot(x, k):
    return pltpu.roll(x, (-k) % x.shape[1], 1)


def _clip01(x):
    return jnp.clip(x, 0.0, 1.0)


def _tree_kernel(x1_ref, x2_ref, ap_ref, ab_ref, w_ref, wz_ref, out_ref):
    f32 = jnp.float32
    bt = x1_ref.shape[0]

    xa = jax.lax.dot_general(
        x2_ref[...], ap_ref[:, 0:512], (((1,), (1,)), ((), ())),
        preferred_element_type=f32) + ab_ref[...]

    lane = jax.lax.broadcasted_iota(jnp.int32, (bt, 128), 1)
    bit = [(lane & (1 << s)) != 0 for s in range(7)]
    lvl = [(lane >= 2 ** d - 1) & (lane < 2 ** (d + 1) - 1) for d in range(7)]

    u = jnp.ones((bt, 128), f32)
    s_group = jnp.zeros((bt, 128), f32)
    for d in range(7):
        s_group = jnp.where(lvl[d], _rot(u, 1), s_group)
        xt = xa[:, d * 128:(d + 1) * 128]
        signed = jnp.where(bit[d], -xt, xt)
        u = jnp.minimum(u, _clip01(signed))
    s_group = jnp.where(lane == 127, 1.0, s_group)
    q7 = u

    xa7 = xa[:, 896:1024]
    xa8 = xa[:, 1024:1280]
    xa9 = xa[:, 1280:1792]
    q8 = jnp.concatenate(
        [jnp.minimum(q7, _clip01(xa7)), jnp.minimum(q7, _clip01(-xa7))], axis=1)
    q9 = jnp.concatenate(
        [jnp.minimum(q8, _clip01(xa8)), jnp.minimum(q8, _clip01(-xa8))], axis=1)
    q10 = jnp.concatenate(
        [jnp.minimum(q9, _clip01(xa9)), jnp.minimum(q9, _clip01(-xa9))], axis=1)
    z = jnp.concatenate([s_group, q7, q8, q9, q10], axis=1)

    out = jax.lax.dot_general(
        x1_ref[...], w_ref[:, 0:512], (((1,), (1,)), ((), ())),
        preferred_element_type=f32)
    out += jax.lax.dot_general(
        z, wz_ref[...], (((1,), (1,)), ((), ())),
        preferred_element_type=f32)
    out_ref[...] = out


@jax.jit
def kernel(X1, X2, A, W):
    batch, in1 = X1.shape
    out_dim = W.shape[0]
    f32 = jnp.float32

    ap = jnp.take(A.astype(f32), _XA_ROW_IDX, axis=0)
    ab = jnp.take(A[:, in1].astype(f32), _XA_ROW_IDX)[None, :]
    wzb = jnp.take(W.astype(f32), _W_COL_IDX, axis=1)

    grid = (batch // _BT,)
    out = pl.pallas_call(
        _tree_kernel,
        grid=grid,
        in_specs=[
            pl.BlockSpec((_BT, in1), lambda i: (i, 0)),
            pl.BlockSpec((_BT, X2.shape[1]), lambda i: (i, 0)),
            pl.BlockSpec((1792, A.shape[1]), lambda i: (0, 0)),
            pl.BlockSpec((1, 1792), lambda i: (0, 0)),
            pl.BlockSpec((out_dim, W.shape[1]), lambda i: (0, 0)),
            pl.BlockSpec((out_dim, 2048), lambda i: (0, 0)),
        ],
        out_specs=pl.BlockSpec((_BT, out_dim), lambda i: (i, 0)),
        out_shape=jax.ShapeDtypeStruct((batch, out_dim), f32),
        compiler_params=pltpu.CompilerParams(
            dimension_semantics=("parallel",)),
    )(X1.astype(f32), X2.astype(f32), ap, ab, W.astype(f32), wzb)
    return out


# speedup vs baseline: 180.3616x; 1.3193x over previous
import jax
import jax.numpy as jnp
import numpy as np
from jax.experimental import pallas as pl
from jax.experimental.pallas import tpu as pltpu

_DEPTH = 10

_LVL = [np.array([0], dtype=np.int64)]
for _d in range(_DEPTH):
    _LVL.append(np.concatenate([2 * _LVL[_d] + 1, 2 * _LVL[_d] + 2]))

_XA_NODES = np.empty(1792, dtype=np.int64)
for _d in range(7):
    _XA_NODES[_d * 128:(_d + 1) * 128] = _LVL[_d][np.arange(128) % (2 ** _d)]
_XA_NODES[896:1024] = _LVL[7]
_XA_NODES[1024:1280] = _LVL[8]
_XA_NODES[1280:1792] = _LVL[9]
_XA_ROW_IDX = _XA_NODES.astype(np.int32)

_Z_NODES = np.full(2048, -1, dtype=np.int64)
for _d in range(7):
    _Z_NODES[2 ** _d - 1: 2 ** (_d + 1) - 1] = _LVL[_d]
_Z_NODES[128:256] = _LVL[7]
_Z_NODES[256:512] = _LVL[8]
_Z_NODES[512:1024] = _LVL[9]
_Z_NODES[1024:2048] = _LVL[10]
_W_COL_IDX = np.where(_Z_NODES >= 0, 513 + _Z_NODES, 512).astype(np.int32)

_PA = np.zeros((1792, 1023), np.float32)
_PA[np.arange(1792), _XA_ROW_IDX] = 1.0
_QW = np.zeros((2560, 2048), np.float32)
_QW[_W_COL_IDX, np.arange(2048)] = 1.0

_BT = 256


def _rot(x, k):
    return pltpu.roll(x, (-k) % x.shape[1], 1)


def _clip01(x):
    return jnp.clip(x, 0.0, 1.0)


def _tree_kernel(x1_ref, x2_ref, a_ref, w_ref, pa_ref, qw_ref, out_ref,
                 ap_ref, ab_ref, wz_ref):
    f32 = jnp.float32
    bf16 = jnp.bfloat16
    bt = x1_ref.shape[0]

    @pl.when(pl.program_id(0) == 0)
    def _prep():
        a_bf = a_ref[...].astype(bf16)
        pa = pa_ref[...]
        ap_ref[...] = jax.lax.dot_general(
            pa, a_bf, (((1,), (0,)), ((), ())), preferred_element_type=f32)
        ab_ref[...] = jax.lax.dot_general(
            a_bf[:, 512:513], pa, (((0,), (1,)), ((), ())),
            preferred_element_type=f32)
        wz_ref[...] = jax.lax.dot_general(
            w_ref[...].astype(bf16), qw_ref[...], (((1,), (0,)), ((), ())),
            preferred_element_type=f32)

    xa = jax.lax.dot_general(
        x2_ref[...], ap_ref[:, 0:512], (((1,), (1,)), ((), ())),
        preferred_element_type=f32) + ab_ref[...]

    lane = jax.lax.broadcasted_iota(jnp.int32, (bt, 128), 1)
    bit = [(lane & (1 << s)) != 0 for s in range(7)]
    lvl = [(lane >= 2 ** d - 1) & (lane < 2 ** (d + 1) - 1) for d in range(7)]

    u = jnp.ones((bt, 128), f32)
    s_group = jnp.zeros((bt, 128), f32)
    for d in range(7):
        s_group = jnp.where(lvl[d], _rot(u, 1), s_group)
        xt = xa[:, d * 128:(d + 1) * 128]
        signed = jnp.where(bit[d], -xt, xt)
        u = jnp.minimum(u, _clip01(signed))
    s_group = jnp.where(lane == 127, 1.0, s_group)
    q7 = u

    xa7 = xa[:, 896:1024]
    xa8 = xa[:, 1024:1280]
    xa9 = xa[:, 1280:1792]
    q8 = jnp.concatenate(
        [jnp.minimum(q7, _clip01(xa7)), jnp.minimum(q7, _clip01(-xa7))], axis=1)
    q9 = jnp.concatenate(
        [jnp.minimum(q8, _clip01(xa8)), jnp.minimum(q8, _clip01(-xa8))], axis=1)
    q10 = jnp.concatenate(
        [jnp.minimum(q9, _clip01(xa9)), jnp.minimum(q9, _clip01(-xa9))], axis=1)
    z = jnp.concatenate([s_group, q7, q8, q9, q10], axis=1)

    out = jax.lax.dot_general(
        x1_ref[...], w_ref[:, 0:512], (((1,), (1,)), ((), ())),
        preferred_element_type=f32)
    out += jax.lax.dot_general(
        z, wz_ref[...], (((1,), (1,)), ((), ())),
        preferred_element_type=f32)
    out_ref[...] = out


@jax.jit
def kernel(X1, X2, A, W):
    batch, in1 = X1.shape
    out_dim = W.shape[0]
    f32 = jnp.float32

    grid = (batch // _BT,)
    out = pl.pallas_call(
        _tree_kernel,
        grid=grid,
        in_specs=[
            pl.BlockSpec((_BT, in1), lambda i: (i, 0)),
            pl.BlockSpec((_BT, X2.shape[1]), lambda i: (i, 0)),
            pl.BlockSpec(A.shape, lambda i: (0, 0)),
            pl.BlockSpec(W.shape, lambda i: (0, 0)),
            pl.BlockSpec(_PA.shape, lambda i: (0, 0)),
            pl.BlockSpec(_QW.shape, lambda i: (0, 0)),
        ],
        out_specs=pl.BlockSpec((_BT, out_dim), lambda i: (i, 0)),
        out_shape=jax.ShapeDtypeStruct((batch, out_dim), f32),
        scratch_shapes=[
            pltpu.VMEM((1792, 513), f32),
            pltpu.VMEM((1, 1792), f32),
            pltpu.VMEM((128, 2048), f32),
        ],
        compiler_params=pltpu.CompilerParams(
            dimension_semantics=("arbitrary",)),
    )(X1.astype(f32), X2.astype(f32), A.astype(f32), W.astype(f32),
      jnp.asarray(_PA, jnp.bfloat16), jnp.asarray(_QW, jnp.bfloat16))
    return out


# speedup vs baseline: 204.5761x; 1.1343x over previous
import jax
import jax.numpy as jnp
import numpy as np
from jax.experimental import pallas as pl
from jax.experimental.pallas import tpu as pltpu

_DEPTH = 10

_LVL = [np.array([0], dtype=np.int64)]
for _d in range(_DEPTH):
    _LVL.append(np.concatenate([2 * _LVL[_d] + 1, 2 * _LVL[_d] + 2]))

_XA_NODES = np.empty(1792, dtype=np.int64)
for _d in range(7):
    _XA_NODES[_d * 128:(_d + 1) * 128] = _LVL[_d][np.arange(128) % (2 ** _d)]
_XA_NODES[896:1024] = _LVL[7]
_XA_NODES[1024:1280] = _LVL[8]
_XA_NODES[1280:1792] = _LVL[9]
_XA_ROW_IDX = _XA_NODES.astype(np.int32)

_Z_NODES = np.full(2048, -1, dtype=np.int64)
for _d in range(7):
    _Z_NODES[2 ** _d - 1: 2 ** (_d + 1) - 1] = _LVL[_d]
_Z_NODES[128:256] = _LVL[7]
_Z_NODES[256:512] = _LVL[8]
_Z_NODES[512:1024] = _LVL[9]
_Z_NODES[1024:2048] = _LVL[10]
_W_COL_IDX = np.where(_Z_NODES >= 0, 513 + _Z_NODES, 512).astype(np.int32)

_PA = np.zeros((1792, 1023), np.float32)
_PA[np.arange(1792), _XA_ROW_IDX] = 1.0
_QW = np.zeros((2560, 2048), np.float32)
_QW[_W_COL_IDX, np.arange(2048)] = 1.0

_BT = 512


def _rot(x, k):
    return pltpu.roll(x, (-k) % x.shape[1], 1)


def _clip01(x):
    return jnp.clip(x, 0.0, 1.0)


def _tree_kernel(x1_ref, x2_ref, a_ref, w_ref, pa_ref, qw_ref, out_ref,
                 ap_ref, ab_ref, wz_ref):
    f32 = jnp.float32
    bf16 = jnp.bfloat16
    bt = x1_ref.shape[0]

    @pl.when(pl.program_id(0) == 0)
    def _prep():
        a_bf = a_ref[...].astype(bf16)
        pa = pa_ref[...]
        ap_ref[...] = jax.lax.dot_general(
            pa, a_bf, (((1,), (0,)), ((), ())), preferred_element_type=f32)
        ab_ref[...] = jax.lax.dot_general(
            a_bf[:, 512:513], pa, (((0,), (1,)), ((), ())),
            preferred_element_type=f32)
        wz_ref[...] = jax.lax.dot_general(
            w_ref[...].astype(bf16), qw_ref[...], (((1,), (0,)), ((), ())),
            preferred_element_type=f32)

    xa = jax.lax.dot_general(
        x2_ref[...], ap_ref[:, 0:512], (((1,), (1,)), ((), ())),
        preferred_element_type=f32) + ab_ref[...]

    lane = jax.lax.broadcasted_iota(jnp.int32, (bt, 128), 1)
    bit = [(lane & (1 << s)) != 0 for s in range(7)]
    lvl = [(lane >= 2 ** d - 1) & (lane < 2 ** (d + 1) - 1) for d in range(7)]

    u = jnp.ones((bt, 128), f32)
    s_group = jnp.zeros((bt, 128), f32)
    for d in range(7):
        s_group = jnp.where(lvl[d], _rot(u, 1), s_group)
        xt = xa[:, d * 128:(d + 1) * 128]
        signed = jnp.where(bit[d], -xt, xt)
        u = jnp.minimum(u, _clip01(signed))
    s_group = jnp.where(lane == 127, 1.0, s_group)
    q7 = u

    xa7 = xa[:, 896:1024]
    xa8 = xa[:, 1024:1280]
    xa9 = xa[:, 1280:1792]
    q8 = jnp.concatenate(
        [jnp.minimum(q7, _clip01(xa7)), jnp.minimum(q7, _clip01(-xa7))], axis=1)
    q9 = jnp.concatenate(
        [jnp.minimum(q8, _clip01(xa8)), jnp.minimum(q8, _clip01(-xa8))], axis=1)
    q10 = jnp.concatenate(
        [jnp.minimum(q9, _clip01(xa9)), jnp.minimum(q9, _clip01(-xa9))], axis=1)
    z = jnp.concatenate([s_group, q7, q8, q9, q10], axis=1)

    out = jax.lax.dot_general(
        x1_ref[...], w_ref[:, 0:512], (((1,), (1,)), ((), ())),
        preferred_element_type=f32)
    out += jax.lax.dot_general(
        z, wz_ref[...], (((1,), (1,)), ((), ())),
        preferred_element_type=f32)
    out_ref[...] = out


@jax.jit
def kernel(X1, X2, A, W):
    batch, in1 = X1.shape
    out_dim = W.shape[0]
    f32 = jnp.float32

    grid = (batch // _BT,)
    out = pl.pallas_call(
        _tree_kernel,
        grid=grid,
        in_specs=[
            pl.BlockSpec((_BT, in1), lambda i: (i, 0)),
            pl.BlockSpec((_BT, X2.shape[1]), lambda i: (i, 0)),
            pl.BlockSpec(A.shape, lambda i: (0, 0)),
            pl.BlockSpec(W.shape, lambda i: (0, 0)),
            pl.BlockSpec(_PA.shape, lambda i: (0, 0)),
            pl.BlockSpec(_QW.shape, lambda i: (0, 0)),
        ],
        out_specs=pl.BlockSpec((_BT, out_dim), lambda i: (i, 0)),
        out_shape=jax.ShapeDtypeStruct((batch, out_dim), f32),
        scratch_shapes=[
            pltpu.VMEM((1792, 513), f32),
            pltpu.VMEM((1, 1792), f32),
            pltpu.VMEM((128, 2048), f32),
        ],
        compiler_params=pltpu.CompilerParams(
            dimension_semantics=("arbitrary",)),
    )(X1.astype(f32), X2.astype(f32), A.astype(f32), W.astype(f32),
      jnp.asarray(_PA, jnp.bfloat16), jnp.asarray(_QW, jnp.bfloat16))
    return out


# speedup vs baseline: 217.7301x; 1.0643x over previous
import jax
import jax.numpy as jnp
import numpy as np
from jax.experimental import pallas as pl
from jax.experimental.pallas import tpu as pltpu

_DEPTH = 10

_LVL = [np.array([0], dtype=np.int64)]
for _d in range(_DEPTH):
    _LVL.append(np.concatenate([2 * _LVL[_d] + 1, 2 * _LVL[_d] + 2]))

_XA_NODES = np.empty(1792, dtype=np.int64)
for _d in range(7):
    _XA_NODES[_d * 128:(_d + 1) * 128] = _LVL[_d][np.arange(128) % (2 ** _d)]
_XA_NODES[896:1024] = _LVL[7]
_XA_NODES[1024:1280] = _LVL[8]
_XA_NODES[1280:1792] = _LVL[9]
_XA_ROW_IDX = _XA_NODES.astype(np.int32)

_Z_NODES = np.full(2048, -1, dtype=np.int64)
for _d in range(7):
    _Z_NODES[2 ** _d - 1: 2 ** (_d + 1) - 1] = _LVL[_d]
_Z_NODES[128:256] = _LVL[7]
_Z_NODES[256:512] = _LVL[8]
_Z_NODES[512:1024] = _LVL[9]
_Z_NODES[1024:2048] = _LVL[10]
_W_COL_IDX = np.where(_Z_NODES >= 0, 513 + _Z_NODES, 512).astype(np.int32)

_PA = np.zeros((1792, 1023), np.float32)
_PA[np.arange(1792), _XA_ROW_IDX] = 1.0
_QW = np.zeros((2560, 2048), np.float32)
_QW[_W_COL_IDX, np.arange(2048)] = 1.0

_BT = 1024


def _rot(x, k):
    return pltpu.roll(x, (-k) % x.shape[1], 1)


def _clip01(x):
    return jnp.clip(x, 0.0, 1.0)


def _tree_kernel(x1_ref, x2_ref, a_ref, w_ref, pa_ref, qw_ref, out_ref,
                 ap_ref, ab_ref, wz_ref):
    f32 = jnp.float32
    bf16 = jnp.bfloat16
    bt = x1_ref.shape[0]

    @pl.when(pl.program_id(0) == 0)
    def _prep():
        a_bf = a_ref[...].astype(bf16)
        pa = pa_ref[...]
        ap_ref[...] = jax.lax.dot_general(
            pa, a_bf, (((1,), (0,)), ((), ())), preferred_element_type=f32)
        ab_ref[...] = jax.lax.dot_general(
            a_bf[:, 512:513], pa, (((0,), (1,)), ((), ())),
            preferred_element_type=f32)
        wz_ref[...] = jax.lax.dot_general(
            w_ref[...].astype(bf16), qw_ref[...], (((1,), (0,)), ((), ())),
            preferred_element_type=f32)

    xa = jax.lax.dot_general(
        x2_ref[...], ap_ref[:, 0:512], (((1,), (1,)), ((), ())),
        preferred_element_type=f32) + ab_ref[...]

    lane = jax.lax.broadcasted_iota(jnp.int32, (bt, 128), 1)
    bit = [(lane & (1 << s)) != 0 for s in range(7)]
    lvl = [(lane >= 2 ** d - 1) & (lane < 2 ** (d + 1) - 1) for d in range(7)]

    u = jnp.ones((bt, 128), f32)
    s_group = jnp.zeros((bt, 128), f32)
    for d in range(7):
        s_group = jnp.where(lvl[d], _rot(u, 1), s_group)
        xt = xa[:, d * 128:(d + 1) * 128]
        signed = jnp.where(bit[d], -xt, xt)
        u = jnp.minimum(u, _clip01(signed))
    s_group = jnp.where(lane == 127, 1.0, s_group)
    q7 = u

    xa7 = xa[:, 896:1024]
    xa8 = xa[:, 1024:1280]
    xa9 = xa[:, 1280:1792]
    q8 = jnp.concatenate(
        [jnp.minimum(q7, _clip01(xa7)), jnp.minimum(q7, _clip01(-xa7))], axis=1)
    q9 = jnp.concatenate(
        [jnp.minimum(q8, _clip01(xa8)), jnp.minimum(q8, _clip01(-xa8))], axis=1)
    q10 = jnp.concatenate(
        [jnp.minimum(q9, _clip01(xa9)), jnp.minimum(q9, _clip01(-xa9))], axis=1)
    z = jnp.concatenate([s_group, q7, q8, q9, q10], axis=1)

    out = jax.lax.dot_general(
        x1_ref[...], w_ref[:, 0:512], (((1,), (1,)), ((), ())),
        preferred_element_type=f32)
    out += jax.lax.dot_general(
        z, wz_ref[...], (((1,), (1,)), ((), ())),
        preferred_element_type=f32)
    out_ref[...] = out


@jax.jit
def kernel(X1, X2, A, W):
    batch, in1 = X1.shape
    out_dim = W.shape[0]
    f32 = jnp.float32

    grid = (batch // _BT,)
    out = pl.pallas_call(
        _tree_kernel,
        grid=grid,
        in_specs=[
            pl.BlockSpec((_BT, in1), lambda i: (i, 0)),
            pl.BlockSpec((_BT, X2.shape[1]), lambda i: (i, 0)),
            pl.BlockSpec(A.shape, lambda i: (0, 0)),
            pl.BlockSpec(W.shape, lambda i: (0, 0)),
            pl.BlockSpec(_PA.shape, lambda i: (0, 0)),
            pl.BlockSpec(_QW.shape, lambda i: (0, 0)),
        ],
        out_specs=pl.BlockSpec((_BT, out_dim), lambda i: (i, 0)),
        out_shape=jax.ShapeDtypeStruct((batch, out_dim), f32),
        scratch_shapes=[
            pltpu.VMEM((1792, 513), f32),
            pltpu.VMEM((1, 1792), f32),
            pltpu.VMEM((128, 2048), f32),
        ],
        compiler_params=pltpu.CompilerParams(
            dimension_semantics=("arbitrary",)),
    )(X1.astype(f32), X2.astype(f32), A.astype(f32), W.astype(f32),
      jnp.asarray(_PA, jnp.bfloat16), jnp.asarray(_QW, jnp.bfloat16))
    return out


# speedup vs baseline: 219.5275x; 1.0083x over previous
import jax
import jax.numpy as jnp
import numpy as np
from jax.experimental import pallas as pl
from jax.experimental.pallas import tpu as pltpu

_DEPTH = 10

_LVL = [np.array([0], dtype=np.int64)]
for _d in range(_DEPTH):
    _LVL.append(np.concatenate([2 * _LVL[_d] + 1, 2 * _LVL[_d] + 2]))

_XA_NODES = np.empty(1792, dtype=np.int64)
for _d in range(7):
    _XA_NODES[_d * 128:(_d + 1) * 128] = _LVL[_d][np.arange(128) % (2 ** _d)]
_XA_NODES[896:1024] = _LVL[7]
_XA_NODES[1024:1280] = _LVL[8]
_XA_NODES[1280:1792] = _LVL[9]
_XA_ROW_IDX = _XA_NODES.astype(np.int32)

_Z_NODES = np.full(2048, -1, dtype=np.int64)
for _d in range(7):
    _Z_NODES[2 ** _d - 1: 2 ** (_d + 1) - 1] = _LVL[_d]
_Z_NODES[128:256] = _LVL[7]
_Z_NODES[256:512] = _LVL[8]
_Z_NODES[512:1024] = _LVL[9]
_Z_NODES[1024:2048] = _LVL[10]
_W_COL_IDX = np.where(_Z_NODES >= 0, 513 + _Z_NODES, 512).astype(np.int32)

_PA = np.zeros((1792, 1023), np.float32)
_PA[np.arange(1792), _XA_ROW_IDX] = 1.0
_QW = np.zeros((2560, 2048), np.float32)
_QW[_W_COL_IDX, np.arange(2048)] = 1.0

_BT = 2048


def _rot(x, k):
    return pltpu.roll(x, (-k) % x.shape[1], 1)


def _clip01(x):
    return jnp.clip(x, 0.0, 1.0)


def _tree_kernel(x1_ref, x2_ref, a_ref, w_ref, pa_ref, qw_ref, out_ref,
                 ap_ref, ab_ref, wz_ref):
    f32 = jnp.float32
    bf16 = jnp.bfloat16
    bt = x1_ref.shape[0]

    @pl.when(pl.program_id(0) == 0)
    def _prep():
        a_bf = a_ref[...].astype(bf16)
        pa = pa_ref[...]
        ap_ref[...] = jax.lax.dot_general(
            pa, a_bf, (((1,), (0,)), ((), ())), preferred_element_type=f32)
        ab_ref[...] = jax.lax.dot_general(
            a_bf[:, 512:513], pa, (((0,), (1,)), ((), ())),
            preferred_element_type=f32)
        wz_ref[...] = jax.lax.dot_general(
            w_ref[...].astype(bf16), qw_ref[...], (((1,), (0,)), ((), ())),
            preferred_element_type=f32)

    xa = jax.lax.dot_general(
        x2_ref[...], ap_ref[:, 0:512], (((1,), (1,)), ((), ())),
        preferred_element_type=f32) + ab_ref[...]

    lane = jax.lax.broadcasted_iota(jnp.int32, (bt, 128), 1)
    bit = [(lane & (1 << s)) != 0 for s in range(7)]
    lvl = [(lane >= 2 ** d - 1) & (lane < 2 ** (d + 1) - 1) for d in range(7)]

    u = jnp.ones((bt, 128), f32)
    s_group = jnp.zeros((bt, 128), f32)
    for d in range(7):
        s_group = jnp.where(lvl[d], _rot(u, 1), s_group)
        xt = xa[:, d * 128:(d + 1) * 128]
        signed = jnp.where(bit[d], -xt, xt)
        u = jnp.minimum(u, _clip01(signed))
    s_group = jnp.where(lane == 127, 1.0, s_group)
    q7 = u

    xa7 = xa[:, 896:1024]
    xa8 = xa[:, 1024:1280]
    xa9 = xa[:, 1280:1792]
    q8 = jnp.concatenate(
        [jnp.minimum(q7, _clip01(xa7)), jnp.minimum(q7, _clip01(-xa7))], axis=1)
    q9 = jnp.concatenate(
        [jnp.minimum(q8, _clip01(xa8)), jnp.minimum(q8, _clip01(-xa8))], axis=1)
    q10 = jnp.concatenate(
        [jnp.minimum(q9, _clip01(xa9)), jnp.minimum(q9, _clip01(-xa9))], axis=1)
    z = jnp.concatenate([s_group, q7, q8, q9, q10], axis=1)

    out = jax.lax.dot_general(
        x1_ref[...], w_ref[:, 0:512], (((1,), (1,)), ((), ())),
        preferred_element_type=f32)
    out += jax.lax.dot_general(
        z, wz_ref[...], (((1,), (1,)), ((), ())),
        preferred_element_type=f32)
    out_ref[...] = out


@jax.jit
def kernel(X1, X2, A, W):
    batch, in1 = X1.shape
    out_dim = W.shape[0]
    f32 = jnp.float32

    grid = (batch // _BT,)
    out = pl.pallas_call(
        _tree_kernel,
        grid=grid,
        in_specs=[
            pl.BlockSpec((_BT, in1), lambda i: (i, 0)),
            pl.BlockSpec((_BT, X2.shape[1]), lambda i: (i, 0)),
            pl.BlockSpec(A.shape, lambda i: (0, 0)),
            pl.BlockSpec(W.shape, lambda i: (0, 0)),
            pl.BlockSpec(_PA.shape, lambda i: (0, 0)),
            pl.BlockSpec(_QW.shape, lambda i: (0, 0)),
        ],
        out_specs=pl.BlockSpec((_BT, out_dim), lambda i: (i, 0)),
        out_shape=jax.ShapeDtypeStruct((batch, out_dim), f32),
        scratch_shapes=[
            pltpu.VMEM((1792, 513), f32),
            pltpu.VMEM((1, 1792), f32),
            pltpu.VMEM((128, 2048), f32),
        ],
        compiler_params=pltpu.CompilerParams(
            dimension_semantics=("arbitrary",)),
    )(X1.astype(f32), X2.astype(f32), A.astype(f32), W.astype(f32),
      jnp.asarray(_PA, jnp.bfloat16), jnp.asarray(_QW, jnp.bfloat16))
    return out
